# Initial kernel scaffold; baseline (speedup 1.0000x reference)
import functools

import jax
import jax.numpy as jnp
from jax import lax
from jax.experimental import pallas as pl
from jax.experimental.pallas import tpu as pltpu
from jax.experimental.pallas import tpu_sc as plsc

N_NODES = 50000
N_EDGES = 800000
EMBED_DIM = 64
NUM_ITERATIONS = 4

L = 16
NS = 16
NC = 2
HALF = EMBED_DIM // NC
BLK = 128
JT = 392
E_PAD = NS * JT * BLK
J1 = JT // NC
N_PAD = 53248
RT = N_PAD // NS
CT = RT // BLK
TRASH = N_PAD - 1


def _edge_sums_body(dstb, wb, a3b, out, dst_v, w_v, ev, zb, a3_v, se):
    c = lax.axis_index("c")
    s = lax.axis_index("s")
    pltpu.sync_copy(dstb.at[s, pl.ds(c * J1, J1)], dst_v)
    pltpu.sync_copy(wb.at[s, pl.ds(c * J1, J1)], w_v)
    pltpu.sync_copy(a3b, a3_v)

    zero16 = jnp.zeros((L,), jnp.float32)

    @pl.loop(0, RT // L)
    def _(i):
        zb[pl.ds(i * L, L)] = zero16

    pltpu.sync_copy(zb, se.at[pl.ds(s * RT, RT)])
    plsc.subcore_barrier()

    a3 = a3_v[...]

    @pl.loop(0, J1)
    def _(j):
        @pl.loop(0, BLK // L)
        def _(i):
            w16 = w_v[j, pl.ds(i * L, L)]
            ev[pl.ds(i * L, L)] = jnp.maximum(a3 * w16, 0.0)

        pltpu.sync_copy(ev, se.at[dst_v.at[j]], add=True)

    plsc.subcore_barrier()
    pltpu.sync_copy(se.at[pl.ds(s * RT, RT)], zb)
    pltpu.sync_copy(zb, out.at[c, pl.ds(s * RT, RT)])


def _step_body(x0, x1, srcb, dstb, biasm, a1b, y0, y1,
               src_v, dst_v, rows0, rows1, a1_v, acc, sg0, sg1):
    c = lax.axis_index("c")
    s = lax.axis_index("s")
    pltpu.sync_copy(srcb.at[s], src_v)
    pltpu.sync_copy(dstb.at[s], dst_v)
    pltpu.sync_copy(a1b, a1_v)

    zero16 = jnp.zeros((L,), jnp.float32)

    @pl.loop(0, BLK)
    def _(i):
        rows0[i, pl.ds(0, L)] = zero16
        rows0[i, pl.ds(L, L)] = zero16

    base = s * RT

    @pl.loop(0, CT)
    def _(k):
        pltpu.sync_copy(rows0, acc.at[pl.ds(base + k * BLK, BLK)])

    plsc.subcore_barrier()

    def main_loop(xh):
        pltpu.async_copy(xh.at[src_v.at[0]], rows0, sg0)
        pltpu.async_copy(xh.at[src_v.at[1]], rows1, sg1)

        @pl.loop(0, JT, step=2)
        def _(j):
            for b, (rows, sem) in enumerate(((rows0, sg0), (rows1, sg1))):
                jj = j + b
                pltpu.make_async_copy(xh.at[src_v.at[jj]], rows, sem).wait()
                pltpu.sync_copy(rows, acc.at[dst_v.at[jj]], add=True)
                nxt = jj + 2

                @pl.when(nxt < JT)
                def _():
                    pltpu.async_copy(xh.at[src_v.at[nxt]], rows, sem)

    @pl.when(c == 0)
    def _():
        main_loop(x0)

    @pl.when(c == 1)
    def _():
        main_loop(x1)

    plsc.subcore_barrier()

    a1 = a1_v[...]

    def final_pass(yh):
        @pl.loop(0, CT)
        def _(k):
            r0 = base + k * BLK
            pltpu.sync_copy(acc.at[pl.ds(r0, BLK)], rows0)
            pltpu.sync_copy(biasm.at[pl.ds(r0, BLK)], rows1)

            @pl.loop(0, BLK)
            def _(i):
                for h in range(2):
                    sl = pl.ds(h * L, L)
                    rows0[i, sl] = jnp.maximum(
                        a1 * rows0[i, sl] + rows1[i, sl], 0.0)

            pltpu.sync_copy(rows0, yh.at[pl.ds(r0, BLK)])

    @pl.when(c == 0)
    def _():
        final_pass(y0)

    @pl.when(c == 1)
    def _():
        final_pass(y1)


_MESH = plsc.VectorSubcoreMesh(core_axis_name="c", subcore_axis_name="s")

_edge_sums_kernel = pl.kernel(
    _edge_sums_body,
    out_type=jax.ShapeDtypeStruct((NC, N_PAD), jnp.float32),
    mesh=_MESH,
    scratch_types=[
        pltpu.VMEM((J1, BLK), jnp.int32),
        pltpu.VMEM((J1, BLK), jnp.float32),
        pltpu.VMEM((BLK,), jnp.float32),
        pltpu.VMEM((RT,), jnp.float32),
        pltpu.VMEM((L,), jnp.float32),
        pltpu.VMEM_SHARED((N_PAD,), jnp.float32),
    ],
)

_step_kernel = pl.kernel(
    _step_body,
    out_type=(jax.ShapeDtypeStruct((N_PAD, HALF), jnp.float32),
              jax.ShapeDtypeStruct((N_PAD, HALF), jnp.float32)),
    mesh=_MESH,
    scratch_types=[
        pltpu.VMEM((JT, BLK), jnp.int32),
        pltpu.VMEM((JT, BLK), jnp.int32),
        pltpu.VMEM((BLK, HALF), jnp.float32),
        pltpu.VMEM((BLK, HALF), jnp.float32),
        pltpu.VMEM((L,), jnp.float32),
        pltpu.VMEM_SHARED((N_PAD, HALF), jnp.float32),
        pltpu.SemaphoreType.DMA,
        pltpu.SemaphoreType.DMA,
    ],
)


@jax.jit
def kernel(x, alphas, edge_weight, labels, edge_index):
    src = edge_index[0].astype(jnp.int32)
    dst = edge_index[1].astype(jnp.int32)
    pad = E_PAD - N_EDGES
    srcb = jnp.pad(src, (0, pad)).reshape(NS, JT, BLK)
    dstb = jnp.pad(dst, (0, pad), constant_values=TRASH).reshape(NS, JT, BLK)
    wb = jnp.pad(edge_weight, (0, pad)).reshape(NS, JT, BLK)

    a1b = jnp.full((L,), alphas[0], jnp.float32)
    a3b = jnp.full((L,), alphas[2], jnp.float32)

    es2 = _edge_sums_kernel(dstb, wb, a3b)
    bias = alphas[1] * (es2[0] + es2[1]) + alphas[3] * jnp.pad(
        labels, (0, N_PAD - N_NODES))
    biasm = jnp.broadcast_to(bias[:, None], (N_PAD, HALF))

    rpad = N_PAD - N_NODES
    x0 = jnp.pad(x[:, :HALF], ((0, rpad), (0, 0)))
    x1 = jnp.pad(x[:, HALF:], ((0, rpad), (0, 0)))
    for _ in range(NUM_ITERATIONS):
        x0, x1 = _step_kernel(x0, x1, srcb, dstb, biasm, a1b)
    return jnp.concatenate([x0[:N_NODES], x1[:N_NODES]], axis=1)


# speedup vs baseline: 1.0813x; 1.0813x over previous
import jax
import jax.numpy as jnp
from jax import lax
from jax.experimental import pallas as pl
from jax.experimental.pallas import tpu as pltpu
from jax.experimental.pallas import tpu_sc as plsc

N_NODES = 50000
N_EDGES = 800000
EMBED_DIM = 64
NUM_ITERATIONS = 4

L = 16
NS = 16
NC = 2
QCOL = 16
NSPLIT = EMBED_DIM // QCOL
BLK = 128
N_PAD = NS * 3328
RANGE = N_PAD // NS
ACC_R = RANGE + BLK
TRASH_L = RANGE
CT = RANGE // BLK
E_CAP = N_EDGES + NS * BLK


def _edge_sums_body(dstlp, wp, meta, a3b, out, dl0, w0, ev, stage, mv, a3_v,
                    se):
    c = lax.axis_index("c")
    s = lax.axis_index("s")

    @pl.when(c == 0)
    def _():
        pltpu.sync_copy(meta, mv)
        pltpu.sync_copy(a3b, a3_v)
        zero16 = jnp.zeros((L,), jnp.float32)

        @pl.loop(0, ACC_R // L)
        def _(i):
            stage[pl.ds(i * L, L)] = zero16

        @pl.loop(0, ACC_R // BLK)
        def _(k):
            pltpu.sync_copy(stage.at[pl.ds(0, BLK)],
                            se.at[pl.ds(k * BLK, BLK)])

        iot = lax.iota(jnp.int32, L)
        nblk = jnp.sum(jnp.where(iot == s, mv[pl.ds(0, L)], 0))
        bblk = jnp.sum(jnp.where(iot == s, mv[pl.ds(L, L)], 0))
        a3 = a3_v[...]

        @pl.loop(0, nblk)
        def _(j):
            off = (bblk + j) * BLK
            pltpu.sync_copy(dstlp.at[pl.ds(off, BLK)], dl0)
            pltpu.sync_copy(wp.at[pl.ds(off, BLK)], w0)

            @pl.loop(0, BLK // L)
            def _(i):
                w16 = w0[pl.ds(i * L, L)]
                ev[pl.ds(i * L, L)] = jnp.maximum(a3 * w16, 0.0)

            pltpu.sync_copy(ev, se.at[dl0], add=True)

        pltpu.sync_copy(se.at[pl.ds(0, RANGE)], stage)
        pltpu.sync_copy(stage, out.at[pl.ds(s * RANGE, RANGE)])


def _embed_body(*refs):
    xs = refs[:NSPLIT]
    srcp, dstlp, meta, biasm, a1b = refs[NSPLIT:NSPLIT + 5]
    ys = refs[NSPLIT + 5:2 * NSPLIT + 5]
    pa = refs[2 * NSPLIT + 5:3 * NSPLIT + 5]
    pb = refs[3 * NSPLIT + 5:4 * NSPLIT + 5]
    (ib0, ib1, db0, db1, rows0, rows1, mv, a1_v, acc,
     si0, si1, sg0, sg1) = refs[4 * NSPLIT + 5:]

    c = lax.axis_index("c")
    s = lax.axis_index("s")
    pltpu.sync_copy(meta, mv)
    pltpu.sync_copy(a1b, a1_v)

    zero16 = jnp.zeros((L,), jnp.float32)
    iot = lax.iota(jnp.int32, L)
    nblk = jnp.sum(jnp.where(iot == s, mv[pl.ds(0, L)], 0))
    bblk = jnp.sum(jnp.where(iot == s, mv[pl.ds(L, L)], 0))
    a1 = a1_v[...]
    obase = s * RANGE
    ib = (ib0, ib1)
    db = (db0, db1)
    rows = (rows0, rows1)
    si = (si0, si1)
    sg = (sg0, sg1)

    def idx_dma(jj, b):
        off = (bblk + jj) * BLK
        pltpu.async_copy(srcp.at[pl.ds(off, BLK)], ib[b], si[b])
        pltpu.async_copy(dstlp.at[pl.ds(off, BLK)], db[b], si[b])

    def idx_wait(jj, b):
        off = (bblk + jj) * BLK
        pltpu.make_async_copy(srcp.at[pl.ds(off, BLK)], ib[b], si[b]).wait()
        pltpu.make_async_copy(dstlp.at[pl.ds(off, BLK)], db[b], si[b]).wait()

    def main_loop(xh):
        @pl.when(nblk > 0)
        def _():
            idx_dma(0, 0)

        @pl.when(nblk > 1)
        def _():
            idx_dma(1, 1)

        @pl.when(nblk > 0)
        def _():
            idx_wait(0, 0)
            pltpu.async_copy(xh.at[ib0], rows0, sg0)

        @pl.loop(0, nblk, step=2)
        def _(j):
            for b in (0, 1):
                jj = j + b

                @pl.when(jj < nblk)
                def _(b=b, jj=jj):
                    pltpu.make_async_copy(xh.at[ib[b]], rows[b],
                                          sg[b]).wait()
                    pltpu.sync_copy(rows[b], acc.at[db[b]], add=True)

                    @pl.when(jj + 2 < nblk)
                    def _():
                        idx_dma(jj + 2, b)

                    @pl.when(jj + 1 < nblk)
                    def _():
                        idx_wait(jj + 1, 1 - b)
                        pltpu.async_copy(xh.at[ib[1 - b]], rows[1 - b],
                                         sg[1 - b])

    def final_pass(yh):
        @pl.loop(0, CT)
        def _(k):
            r0 = obase + k * BLK
            pltpu.sync_copy(acc.at[pl.ds(k * BLK, BLK)], rows0)
            pltpu.sync_copy(biasm.at[pl.ds(r0, BLK)], rows1)

            @pl.loop(0, BLK)
            def _(i):
                sl = pl.ds(0, L)
                rows0[i, sl] = jnp.maximum(
                    a1 * rows0[i, sl] + rows1[i, sl], 0.0)

            pltpu.sync_copy(rows0, yh.at[pl.ds(r0, BLK)])

    reads = [xs, pa, pb, pa]
    writes = [pa, pb, pa, ys]

    for it in range(NUM_ITERATIONS):
        for p in range(NSPLIT // NC):
            @pl.loop(0, BLK)
            def _(i):
                rows0[i, pl.ds(0, L)] = zero16

            @pl.loop(0, ACC_R // BLK)
            def _(k):
                pltpu.sync_copy(rows0, acc.at[pl.ds(k * BLK, BLK)])

            for ci in range(NC):
                q = p * NC + ci

                @pl.when(c == ci)
                def _(q=q):
                    main_loop(reads[it][q])
                    final_pass(writes[it][q])

        plsc.subcore_barrier()


_MESH = plsc.VectorSubcoreMesh(core_axis_name="c", subcore_axis_name="s")

_edge_sums_kernel = pl.kernel(
    _edge_sums_body,
    out_type=jax.ShapeDtypeStruct((N_PAD,), jnp.float32),
    mesh=_MESH,
    compiler_params=pltpu.CompilerParams(use_tc_tiling_on_sc=False, needs_layout_passes=False),
    scratch_types=[
        pltpu.VMEM((BLK,), jnp.int32),
        pltpu.VMEM((BLK,), jnp.float32),
        pltpu.VMEM((BLK,), jnp.float32),
        pltpu.VMEM((RANGE,), jnp.float32),
        pltpu.VMEM((2 * L,), jnp.int32),
        pltpu.VMEM((L,), jnp.float32),
        pltpu.VMEM_SHARED((ACC_R,), jnp.float32),
    ],
)

_SLICE_T = jax.ShapeDtypeStruct((N_PAD, QCOL), jnp.float32)

_embed_kernel = pl.kernel(
    _embed_body,
    out_type=tuple(_SLICE_T for _ in range(3 * NSPLIT)),
    mesh=_MESH,
    compiler_params=pltpu.CompilerParams(use_tc_tiling_on_sc=False, needs_layout_passes=False),
    scratch_types=[
        pltpu.VMEM((BLK,), jnp.int32),
        pltpu.VMEM((BLK,), jnp.int32),
        pltpu.VMEM((BLK,), jnp.int32),
        pltpu.VMEM((BLK,), jnp.int32),
        pltpu.VMEM((BLK, QCOL), jnp.float32),
        pltpu.VMEM((BLK, QCOL), jnp.float32),
        pltpu.VMEM((2 * L,), jnp.int32),
        pltpu.VMEM((L,), jnp.float32),
        pltpu.VMEM_SHARED((ACC_R, QCOL), jnp.float32),
        pltpu.SemaphoreType.DMA,
        pltpu.SemaphoreType.DMA,
        pltpu.SemaphoreType.DMA,
        pltpu.SemaphoreType.DMA,
    ],
)


@jax.jit
def kernel(x, alphas, edge_weight, labels, edge_index):
    src = edge_index[0].astype(jnp.int32)
    dst = edge_index[1].astype(jnp.int32)

    order = jnp.argsort(dst)
    dst_s = dst[order]
    src_s = src[order]
    w_s = edge_weight[order].astype(jnp.float32)
    bounds = jnp.searchsorted(dst_s, RANGE * jnp.arange(NS + 1),
                              side="left").astype(jnp.int32)
    counts = bounds[1:] - bounds[:-1]
    pcounts = ((counts + (BLK - 1)) // BLK) * BLK
    starts = jnp.concatenate(
        [jnp.zeros((1,), jnp.int32), jnp.cumsum(pcounts)[:-1]])
    bkt = dst_s // RANGE
    slot = starts[bkt] + (jnp.arange(N_EDGES, dtype=jnp.int32) - bounds[bkt])
    srcp = jnp.zeros((E_CAP,), jnp.int32).at[slot].set(src_s)
    dstlp = jnp.full((E_CAP,), TRASH_L, jnp.int32).at[slot].set(
        dst_s - bkt * RANGE)
    wp = jnp.zeros((E_CAP,), jnp.float32).at[slot].set(w_s)
    meta = jnp.concatenate([pcounts // BLK, starts // BLK])

    a1b = jnp.full((L,), alphas[0], jnp.float32)
    a3b = jnp.full((L,), alphas[2], jnp.float32)

    es = _edge_sums_kernel(dstlp, wp, meta, a3b)
    bias = alphas[1] * es + alphas[3] * jnp.pad(
        labels, (0, N_PAD - N_NODES))
    biasm = jnp.broadcast_to(bias[:, None], (N_PAD, QCOL))

    rpad = N_PAD - N_NODES
    xq = [jnp.pad(x[:, q * QCOL:(q + 1) * QCOL], ((0, rpad), (0, 0)))
          for q in range(NSPLIT)]
    outs = _embed_kernel(*xq, srcp, dstlp, meta, biasm, a1b)
    return jnp.concatenate([outs[q][:N_NODES] for q in range(NSPLIT)], axis=1)


# speedup vs baseline: 2.8261x; 2.6136x over previous
import jax
import jax.numpy as jnp
from jax import lax
from jax.experimental import pallas as pl
from jax.experimental.pallas import tpu as pltpu
from jax.experimental.pallas import tpu_sc as plsc

N_NODES = 50000
N_EDGES = 800000
EMBED_DIM = 64
NUM_ITERATIONS = 4

L = 16
NS = 16
NC = 2
QCOL = 16
NSPLIT = EMBED_DIM // QCOL
BLK = 128
N_PAD = NS * 3328
RANGE = N_PAD // NS
ACC_R = RANGE + BLK
TRASH_L = RANGE
CT = RANGE // BLK
E_CAP = N_EDGES + NS * BLK


def _edge_sums_body(dstlp, wp, meta, a3b, out, dl0, w0, ev, stage, mv, a3_v,
                    se):
    c = lax.axis_index("c")
    s = lax.axis_index("s")

    @pl.when(c == 0)
    def _():
        pltpu.sync_copy(meta, mv)
        pltpu.sync_copy(a3b, a3_v)
        zero16 = jnp.zeros((L,), jnp.float32)

        @pl.loop(0, ACC_R // L)
        def _(i):
            stage[pl.ds(i * L, L)] = zero16

        @pl.loop(0, ACC_R // BLK)
        def _(k):
            pltpu.sync_copy(stage.at[pl.ds(0, BLK)],
                            se.at[pl.ds(k * BLK, BLK)])

        iot = lax.iota(jnp.int32, L)
        nblk = jnp.sum(jnp.where(iot == s, mv[pl.ds(0, L)], 0))
        bblk = jnp.sum(jnp.where(iot == s, mv[pl.ds(L, L)], 0))
        a3 = a3_v[...]

        @pl.loop(0, nblk)
        def _(j):
            off = (bblk + j) * BLK
            pltpu.sync_copy(dstlp.at[pl.ds(off, BLK)], dl0)
            pltpu.sync_copy(wp.at[pl.ds(off, BLK)], w0)

            @pl.loop(0, BLK // L)
            def _(i):
                w16 = w0[pl.ds(i * L, L)]
                ev[pl.ds(i * L, L)] = jnp.maximum(a3 * w16, 0.0)

            pltpu.sync_copy(ev, se.at[dl0], add=True)

        pltpu.sync_copy(se.at[pl.ds(0, RANGE)], stage)
        pltpu.sync_copy(stage, out.at[pl.ds(s * RANGE, RANGE)])


def _embed_body(*refs):
    xs = refs[:NSPLIT]
    srcp, dstlp, meta, biasm, a1b = refs[NSPLIT:NSPLIT + 5]
    ys = refs[NSPLIT + 5:2 * NSPLIT + 5]
    pa = refs[2 * NSPLIT + 5:3 * NSPLIT + 5]
    pb = refs[3 * NSPLIT + 5:4 * NSPLIT + 5]
    (ib0, ib1, db0, db1, rows0, rows1, mv, a1_v, acc,
     si0, si1, sg0, sg1) = refs[4 * NSPLIT + 5:]

    c = lax.axis_index("c")
    s = lax.axis_index("s")
    pltpu.sync_copy(meta, mv)
    pltpu.sync_copy(a1b, a1_v)

    zero16 = jnp.zeros((L,), jnp.float32)
    iot = lax.iota(jnp.int32, L)
    nblk = jnp.sum(jnp.where(iot == s, mv[pl.ds(0, L)], 0))
    bblk = jnp.sum(jnp.where(iot == s, mv[pl.ds(L, L)], 0))
    a1 = a1_v[...]
    obase = s * RANGE
    ib = (ib0, ib1)
    db = (db0, db1)
    rows = (rows0, rows1)
    si = (si0, si1)
    sg = (sg0, sg1)

    def idx_dma(jj, b):
        off = (bblk + jj) * BLK
        pltpu.async_copy(srcp.at[pl.ds(off, BLK)], ib[b], si[b])
        pltpu.async_copy(dstlp.at[pl.ds(off, BLK)], db[b], si[b])

    def idx_wait(jj, b):
        off = (bblk + jj) * BLK
        pltpu.make_async_copy(srcp.at[pl.ds(off, BLK)], ib[b], si[b]).wait()
        pltpu.make_async_copy(dstlp.at[pl.ds(off, BLK)], db[b], si[b]).wait()

    def main_loop(xh):
        @pl.when(nblk > 0)
        def _():
            idx_dma(0, 0)

        @pl.when(nblk > 1)
        def _():
            idx_dma(1, 1)

        @pl.when(nblk > 0)
        def _():
            idx_wait(0, 0)
            pltpu.async_copy(xh.at[ib0], rows0, sg0)

        @pl.loop(0, nblk, step=2)
        def _(j):
            for b in (0, 1):
                jj = j + b

                @pl.when(jj < nblk)
                def _(b=b, jj=jj):
                    pltpu.make_async_copy(xh.at[ib[b]], rows[b],
                                          sg[b]).wait()
                    pltpu.sync_copy(rows[b], acc.at[db[b]], add=True)

                    @pl.when(jj + 2 < nblk)
                    def _():
                        idx_dma(jj + 2, b)

                    @pl.when(jj + 1 < nblk)
                    def _():
                        idx_wait(jj + 1, 1 - b)
                        pltpu.async_copy(xh.at[ib[1 - b]], rows[1 - b],
                                         sg[1 - b])

    def final_pass(yh):
        @pl.loop(0, CT)
        def _(k):
            r0 = obase + k * BLK
            pltpu.sync_copy(acc.at[pl.ds(k * BLK, BLK)], rows0)
            pltpu.sync_copy(biasm.at[pl.ds(r0, BLK)], rows1)

            @pl.loop(0, BLK)
            def _(i):
                sl = pl.ds(0, L)
                rows0[i, sl] = jnp.maximum(
                    a1 * rows0[i, sl] + rows1[i, sl], 0.0)

            pltpu.sync_copy(rows0, yh.at[pl.ds(r0, BLK)])

    reads = [xs, pa, pb, pa]
    writes = [pa, pb, pa, ys]

    for it in range(NUM_ITERATIONS):
        for p in range(NSPLIT // NC):
            @pl.loop(0, BLK)
            def _(i):
                rows0[i, pl.ds(0, L)] = zero16

            @pl.loop(0, ACC_R // BLK)
            def _(k):
                pltpu.sync_copy(rows0, acc.at[pl.ds(k * BLK, BLK)])

            for ci in range(NC):
                q = p * NC + ci

                @pl.when(c == ci)
                def _(q=q):
                    main_loop(reads[it][q])
                    final_pass(writes[it][q])

        plsc.subcore_barrier()


_MESH = plsc.VectorSubcoreMesh(core_axis_name="c", subcore_axis_name="s")

_edge_sums_kernel = pl.kernel(
    _edge_sums_body,
    out_type=jax.ShapeDtypeStruct((N_PAD,), jnp.float32),
    mesh=_MESH,
    compiler_params=pltpu.CompilerParams(use_tc_tiling_on_sc=False, needs_layout_passes=False),
    scratch_types=[
        pltpu.VMEM((BLK,), jnp.int32),
        pltpu.VMEM((BLK,), jnp.float32),
        pltpu.VMEM((BLK,), jnp.float32),
        pltpu.VMEM((RANGE,), jnp.float32),
        pltpu.VMEM((2 * L,), jnp.int32),
        pltpu.VMEM((L,), jnp.float32),
        pltpu.VMEM_SHARED((ACC_R,), jnp.float32),
    ],
)

_SLICE_T = jax.ShapeDtypeStruct((N_PAD, QCOL), jnp.float32)

_embed_kernel = pl.kernel(
    _embed_body,
    out_type=tuple(_SLICE_T for _ in range(3 * NSPLIT)),
    mesh=_MESH,
    compiler_params=pltpu.CompilerParams(use_tc_tiling_on_sc=False, needs_layout_passes=False),
    scratch_types=[
        pltpu.VMEM((BLK,), jnp.int32),
        pltpu.VMEM((BLK,), jnp.int32),
        pltpu.VMEM((BLK,), jnp.int32),
        pltpu.VMEM((BLK,), jnp.int32),
        pltpu.VMEM((BLK, QCOL), jnp.float32),
        pltpu.VMEM((BLK, QCOL), jnp.float32),
        pltpu.VMEM((2 * L,), jnp.int32),
        pltpu.VMEM((L,), jnp.float32),
        pltpu.VMEM_SHARED((ACC_R, QCOL), jnp.float32),
        pltpu.SemaphoreType.DMA,
        pltpu.SemaphoreType.DMA,
        pltpu.SemaphoreType.DMA,
        pltpu.SemaphoreType.DMA,
    ],
)


@jax.jit
def kernel(x, alphas, edge_weight, labels, edge_index):
    src = edge_index[0].astype(jnp.int32)
    dst = edge_index[1].astype(jnp.int32)

    order = jnp.argsort(dst)
    dst_s = dst[order]
    src_s = src[order]
    w_s = edge_weight[order].astype(jnp.float32)
    bounds = jnp.searchsorted(dst_s, RANGE * jnp.arange(NS + 1),
                              side="left").astype(jnp.int32)
    counts = bounds[1:] - bounds[:-1]
    pcounts = ((counts + (BLK - 1)) // BLK) * BLK
    starts = jnp.concatenate(
        [jnp.zeros((1,), jnp.int32), jnp.cumsum(pcounts)[:-1]])
    t = jnp.arange(E_CAP, dtype=jnp.int32)
    bt = jnp.searchsorted(starts, t, side="right").astype(jnp.int32) - 1
    fillpos = t - starts[bt]
    filled = fillpos < counts[bt]
    gidx = jnp.minimum(bounds[bt] + fillpos, N_EDGES - 1)
    srcp = src_s[gidx]
    dstlp = jnp.where(filled, dst_s[gidx] - bt * RANGE, TRASH_L)
    wp = jnp.where(filled, w_s[gidx], 0.0)
    meta = jnp.concatenate([pcounts // BLK, starts // BLK])

    a1b = jnp.full((L,), alphas[0], jnp.float32)
    a3b = jnp.full((L,), alphas[2], jnp.float32)

    es = _edge_sums_kernel(dstlp, wp, meta, a3b)
    bias = alphas[1] * es + alphas[3] * jnp.pad(
        labels, (0, N_PAD - N_NODES))
    biasm = jnp.broadcast_to(bias[:, None], (N_PAD, QCOL))

    rpad = N_PAD - N_NODES
    xq = [jnp.pad(x[:, q * QCOL:(q + 1) * QCOL], ((0, rpad), (0, 0)))
          for q in range(NSPLIT)]
    outs = _embed_kernel(*xq, srcp, dstlp, meta, biasm, a1b)
    return jnp.concatenate([outs[q][:N_NODES] for q in range(NSPLIT)], axis=1)
